# baseline (speedup 1.0000x reference)
import functools

import jax
import jax.numpy as jnp
from jax import lax
from jax.experimental import pallas as pl
from jax.experimental.pallas import tpu as pltpu
from jax.experimental.pallas import tpu_sc as plsc

VOCAB = 100000
EMB = 64
BATCH = 4096
SEQ = 50
NC, NS = 2, 16
NW = NC * NS
B_PER_W = BATCH // NW
B_CHUNK = 2
N_CHUNKS = B_PER_W // B_CHUNK
NBUF = 4


def _sc_lookup(table, seq):
    mesh = plsc.VectorSubcoreMesh(
        core_axis_name="c", subcore_axis_name="s",
        num_cores=NC, num_subcores=NS)

    @functools.partial(
        pl.kernel,
        out_type=jax.ShapeDtypeStruct((BATCH, SEQ, EMB), jnp.float32),
        mesh=mesh,
        scratch_types=[
            pltpu.VMEM((N_CHUNKS, B_CHUNK * SEQ), jnp.int32),
            [pltpu.VMEM((B_CHUNK * SEQ, EMB), jnp.float32)
             for _ in range(NBUF)],
            [pltpu.SemaphoreType.DMA for _ in range(NBUF)],
        ],
        compiler_params=pltpu.CompilerParams(use_tc_tiling_on_sc=False),
    )
    def k(table_hbm, seq_hbm, out_hbm, idx_v, rows, gsems):
        wid = lax.axis_index("s") * NC + lax.axis_index("c")
        b0 = wid * B_PER_W
        pltpu.sync_copy(seq_hbm.at[wid], idx_v)

        for b in range(NBUF):
            pltpu.async_copy(table_hbm.at[idx_v.at[b]], rows[b], gsems[b])

        def outer(g, carry):
            for b in range(NBUF):
                c = g * NBUF + b
                pltpu.make_async_copy(
                    table_hbm.at[idx_v.at[c]], rows[b], gsems[b]).wait()
                for j in range(B_CHUNK):
                    pltpu.sync_copy(
                        rows[b].at[pl.ds(j * SEQ, SEQ)],
                        out_hbm.at[b0 + c * B_CHUNK + j])
                nxt = c + NBUF

                @pl.when(nxt < N_CHUNKS)
                def _():
                    pltpu.async_copy(
                        table_hbm.at[idx_v.at[nxt]], rows[b], gsems[b])
            return carry

        lax.fori_loop(0, N_CHUNKS // NBUF, outer, 0)

    return k(table, seq)


def kernel(seq, W):
    seq2 = seq.reshape(NW, N_CHUNKS, B_CHUNK * SEQ).astype(jnp.int32)
    return _sc_lookup(W, seq2)
